# initial kernel scaffold (unmeasured)
import jax
import jax.numpy as jnp
from jax import lax
from jax.experimental import pallas as pl
from jax.experimental.pallas import tpu as pltpu

N_DEV = 32
N_STEPS = 2 * (N_DEV - 1)


def _ring_allreduce(x, *, collective_id):
    rows, cols = x.shape
    chunk = rows // N_DEV

    def body(x_ref, out_ref, comm_ref, send_sems, recv_sems):
        my = lax.axis_index("i")
        left = (my - 1) % N_DEV
        right = (my + 1) % N_DEV

        barrier = pltpu.get_barrier_semaphore()
        pl.semaphore_signal(
            barrier, inc=1, device_id=(left,),
            device_id_type=pl.DeviceIdType.MESH,
        )
        pl.semaphore_signal(
            barrier, inc=1, device_id=(right,),
            device_id_type=pl.DeviceIdType.MESH,
        )
        pl.semaphore_wait(barrier, 2)

        out_ref[...] = x_ref[...]

        for s in range(N_DEV - 1):
            c_send = (my - s) % N_DEV
            rdma = pltpu.make_async_remote_copy(
                src_ref=out_ref.at[pl.ds(c_send * chunk, chunk), :],
                dst_ref=comm_ref.at[s],
                send_sem=send_sems.at[s],
                recv_sem=recv_sems.at[s],
                device_id=(right,),
                device_id_type=pl.DeviceIdType.MESH,
            )
            rdma.start()
            rdma.wait()
            c_recv = (my - s - 1) % N_DEV
            idx = (pl.ds(c_recv * chunk, chunk), slice(None))
            pl.store(out_ref, idx, pl.load(out_ref, idx) + comm_ref[s])

        for j in range(N_DEV - 1):
            s = (N_DEV - 1) + j
            c_send = (my + 1 - j) % N_DEV
            rdma = pltpu.make_async_remote_copy(
                src_ref=out_ref.at[pl.ds(c_send * chunk, chunk), :],
                dst_ref=comm_ref.at[s],
                send_sem=send_sems.at[s],
                recv_sem=recv_sems.at[s],
                device_id=(right,),
                device_id_type=pl.DeviceIdType.MESH,
            )
            rdma.start()
            rdma.wait()
            c_recv = (my - j) % N_DEV
            idx = (pl.ds(c_recv * chunk, chunk), slice(None))
            pl.store(out_ref, idx, comm_ref[s])

    return pl.pallas_call(
        body,
        out_shape=jax.ShapeDtypeStruct((rows, cols), x.dtype),
        in_specs=[pl.BlockSpec(memory_space=pltpu.VMEM)],
        out_specs=pl.BlockSpec(memory_space=pltpu.VMEM),
        scratch_shapes=[
            pltpu.VMEM((N_STEPS, chunk, cols), x.dtype),
            pltpu.SemaphoreType.DMA((N_STEPS,)),
            pltpu.SemaphoreType.DMA((N_STEPS,)),
        ],
        compiler_params=pltpu.CompilerParams(collective_id=collective_id),
    )(x)


def kernel(x, Wq, Wk, Wv, Wo, t_emb, W_mod, W_ff1, W_ff2):
    B, S, D = x.shape
    eps = 1e-5
    Dh = 128
    H = Wq.shape[1] // Dh

    mod = t_emb @ W_mod
    sa, sha, ga, sm, shm, gm = jnp.split(mod, 6, axis=-1)

    def ln(h):
        m = h.mean(axis=-1, keepdims=True)
        v = h.var(axis=-1, keepdims=True)
        return (h - m) * lax.rsqrt(v + eps)

    x0 = x
    xm = ln(x0) * (1.0 + sa[:, None, :]) + sha[:, None, :]

    Q = (xm @ Wq).reshape(B, S, H, Dh)
    K = (xm @ Wk).reshape(B, S, H, Dh)
    V = (xm @ Wv).reshape(B, S, H, Dh)
    scores = jnp.einsum("bihd,bjhd->bhij", Q, K) * 0.08838834764831843
    p = jax.nn.softmax(scores, axis=-1)
    o = jnp.einsum("bhij,bjhd->bihd", p, V).reshape(B, S, H * Dh)
    attn_partial = o @ Wo

    attn = _ring_allreduce(
        attn_partial.reshape(B * S, D), collective_id=0
    ).reshape(B, S, D)

    x1 = x0 + ga[:, None, :] * attn
    xm2 = ln(x1) * (1.0 + sm[:, None, :]) + shm[:, None, :]
    h = xm2 @ W_ff1
    h = h * jax.nn.sigmoid(h)
    ffn_partial = h @ W_ff2

    ffn = _ring_allreduce(
        ffn_partial.reshape(B * S, D), collective_id=1
    ).reshape(B, S, D)

    return x1 + gm[:, None, :] * ffn


# baseline (device time: 709334 ns/iter reference)
import jax
import jax.numpy as jnp
from jax import lax
from jax.experimental import pallas as pl
from jax.experimental.pallas import tpu as pltpu

N_DEV = 32
N_STEPS = 2 * (N_DEV - 1)


def _ring_allreduce(x, *, collective_id):
    rows, cols = x.shape
    chunk = rows // N_DEV

    def body(x_ref, out_ref, comm_ref, send_sems, recv_sems):
        my = lax.axis_index("i")
        left = (my - 1) % N_DEV
        right = (my + 1) % N_DEV

        barrier = pltpu.get_barrier_semaphore()
        pl.semaphore_signal(
            barrier, inc=1, device_id=(left,),
            device_id_type=pl.DeviceIdType.MESH,
        )
        pl.semaphore_signal(
            barrier, inc=1, device_id=(right,),
            device_id_type=pl.DeviceIdType.MESH,
        )
        pl.semaphore_wait(barrier, 2)

        out_ref[...] = x_ref[...]

        for s in range(N_DEV - 1):
            c_send = (my - s) % N_DEV
            rdma = pltpu.make_async_remote_copy(
                src_ref=out_ref.at[pl.ds(c_send * chunk, chunk), :],
                dst_ref=comm_ref.at[s],
                send_sem=send_sems.at[s],
                recv_sem=recv_sems.at[s],
                device_id=(right,),
                device_id_type=pl.DeviceIdType.MESH,
            )
            rdma.start()
            rdma.wait()
            c_recv = (my - s - 1) % N_DEV
            idx = (pl.ds(c_recv * chunk, chunk), slice(None))
            out_ref[idx] = out_ref[idx] + comm_ref[s]

        for j in range(N_DEV - 1):
            s = (N_DEV - 1) + j
            c_send = (my + 1 - j) % N_DEV
            rdma = pltpu.make_async_remote_copy(
                src_ref=out_ref.at[pl.ds(c_send * chunk, chunk), :],
                dst_ref=comm_ref.at[s],
                send_sem=send_sems.at[s],
                recv_sem=recv_sems.at[s],
                device_id=(right,),
                device_id_type=pl.DeviceIdType.MESH,
            )
            rdma.start()
            rdma.wait()
            c_recv = (my - j) % N_DEV
            idx = (pl.ds(c_recv * chunk, chunk), slice(None))
            out_ref[idx] = comm_ref[s]

    return pl.pallas_call(
        body,
        out_shape=jax.ShapeDtypeStruct((rows, cols), x.dtype),
        in_specs=[pl.BlockSpec(memory_space=pltpu.VMEM)],
        out_specs=pl.BlockSpec(memory_space=pltpu.VMEM),
        scratch_shapes=[
            pltpu.VMEM((N_STEPS, chunk, cols), x.dtype),
            pltpu.SemaphoreType.DMA((N_STEPS,)),
            pltpu.SemaphoreType.DMA((N_STEPS,)),
        ],
        compiler_params=pltpu.CompilerParams(collective_id=collective_id),
    )(x)


def kernel(x, Wq, Wk, Wv, Wo, t_emb, W_mod, W_ff1, W_ff2):
    B, S, D = x.shape
    eps = 1e-5
    Dh = 128
    H = Wq.shape[1] // Dh

    mod = t_emb @ W_mod
    sa, sha, ga, sm, shm, gm = jnp.split(mod, 6, axis=-1)

    def ln(h):
        m = h.mean(axis=-1, keepdims=True)
        v = h.var(axis=-1, keepdims=True)
        return (h - m) * lax.rsqrt(v + eps)

    x0 = x
    xm = ln(x0) * (1.0 + sa[:, None, :]) + sha[:, None, :]

    Q = (xm @ Wq).reshape(B, S, H, Dh)
    K = (xm @ Wk).reshape(B, S, H, Dh)
    V = (xm @ Wv).reshape(B, S, H, Dh)
    scores = jnp.einsum("bihd,bjhd->bhij", Q, K) * 0.08838834764831843
    p = jax.nn.softmax(scores, axis=-1)
    o = jnp.einsum("bhij,bjhd->bihd", p, V).reshape(B, S, H * Dh)
    attn_partial = o @ Wo

    attn = _ring_allreduce(
        attn_partial.reshape(B * S, D), collective_id=0
    ).reshape(B, S, D)

    x1 = x0 + ga[:, None, :] * attn
    xm2 = ln(x1) * (1.0 + sm[:, None, :]) + shm[:, None, :]
    h = xm2 @ W_ff1
    h = h * jax.nn.sigmoid(h)
    ffn_partial = h @ W_ff2

    ffn = _ring_allreduce(
        ffn_partial.reshape(B * S, D), collective_id=1
    ).reshape(B, S, D)

    return x1 + gm[:, None, :] * ffn


# device time: 392482 ns/iter; 1.8073x vs baseline; 1.8073x over previous
import jax
import jax.numpy as jnp
from jax import lax
from jax.experimental import pallas as pl
from jax.experimental.pallas import tpu as pltpu

N_DEV = 32
PLANE = 8
NZ = 4

Q_TABLE = (0, 1, 2, 7, 6, 3, 4, 5)
INV_TABLE = (0, 1, 2, 5, 6, 7, 4, 3)


def _lut(idx, table):
    out = jnp.int32(table[0])
    for i, v in enumerate(table[1:], start=1):
        out = jnp.where(idx == i, jnp.int32(v), out)
    return out


def _hier_allreduce(x, *, collective_id):
    rows, cols = x.shape
    half = rows // 2
    ch = half // PLANE
    sub = ch // NZ

    def body(x_ref, out_ref, comm1, comm2, comm3,
             s1s, r1s, s2s, r2s, s3s, r3s):
        k = lax.axis_index("i")
        z = k // PLANE
        p = k % PLANE
        q = _lut(p, Q_TABLE)
        plane_base = z * PLANE
        nxt = plane_base + _lut((q + 1) % PLANE, INV_TABLE)
        prv = plane_base + _lut((q + 7) % PLANE, INV_TABLE)
        up = ((z + 1) % NZ) * PLANE + p
        dn = ((z + 3) % NZ) * PLANE + p

        barrier = pltpu.get_barrier_semaphore()
        for nbr in (nxt, prv, up, dn):
            pl.semaphore_signal(
                barrier, inc=1, device_id=(nbr,),
                device_id_type=pl.DeviceIdType.MESH,
            )
        pl.semaphore_wait(barrier, 4)

        out_ref[...] = x_ref[...]

        def chunk_rows(d, c):
            return pl.ds(d * half + c * ch, ch)

        def sub_rows(d, own_c, si):
            return pl.ds(d * half + own_c * ch + si * sub, sub)

        for s in range(PLANE - 1):
            rdmas = []
            for d in (0, 1):
                c_send = (q - s) % PLANE if d == 0 else (q + s) % PLANE
                tgt = nxt if d == 0 else prv
                rdma = pltpu.make_async_remote_copy(
                    src_ref=out_ref.at[chunk_rows(d, c_send), :],
                    dst_ref=comm1.at[d, s],
                    send_sem=s1s.at[d, s],
                    recv_sem=r1s.at[d, s],
                    device_id=(tgt,),
                    device_id_type=pl.DeviceIdType.MESH,
                )
                rdma.start()
                rdmas.append(rdma)
            for d, rdma in enumerate(rdmas):
                rdma.wait()
                c_recv = (q - s - 1) % PLANE if d == 0 else (q + s + 1) % PLANE
                idx = (chunk_rows(d, c_recv), slice(None))
                out_ref[idx] = out_ref[idx] + comm1[d, s]

        own = ((q + 1) % PLANE, (q + 7) % PLANE)

        for s in range(NZ - 1):
            rdmas = []
            for d in (0, 1):
                si_send = (z - s) % NZ if d == 0 else (z + s) % NZ
                tgt = up if d == 0 else dn
                rdma = pltpu.make_async_remote_copy(
                    src_ref=out_ref.at[sub_rows(d, own[d], si_send), :],
                    dst_ref=comm2.at[d, s],
                    send_sem=s2s.at[d, s],
                    recv_sem=r2s.at[d, s],
                    device_id=(tgt,),
                    device_id_type=pl.DeviceIdType.MESH,
                )
                rdma.start()
                rdmas.append(rdma)
            for d, rdma in enumerate(rdmas):
                rdma.wait()
                si_recv = (z - s - 1) % NZ if d == 0 else (z + s + 1) % NZ
                idx = (sub_rows(d, own[d], si_recv), slice(None))
                out_ref[idx] = out_ref[idx] + comm2[d, s]

        for j in range(NZ - 1):
            rdmas = []
            for d in (0, 1):
                si_send = (z + 1 - j) % NZ if d == 0 else (z - 1 + j) % NZ
                tgt = up if d == 0 else dn
                rdma = pltpu.make_async_remote_copy(
                    src_ref=out_ref.at[sub_rows(d, own[d], si_send), :],
                    dst_ref=comm2.at[d, (NZ - 1) + j],
                    send_sem=s2s.at[d, (NZ - 1) + j],
                    recv_sem=r2s.at[d, (NZ - 1) + j],
                    device_id=(tgt,),
                    device_id_type=pl.DeviceIdType.MESH,
                )
                rdma.start()
                rdmas.append(rdma)
            for d, rdma in enumerate(rdmas):
                rdma.wait()
                si_recv = (z - j) % NZ if d == 0 else (z + j) % NZ
                idx = (sub_rows(d, own[d], si_recv), slice(None))
                out_ref[idx] = comm2[d, (NZ - 1) + j]

        for j in range(PLANE - 1):
            rdmas = []
            for d in (0, 1):
                c_send = (q + 1 - j) % PLANE if d == 0 else (q + 7 + j) % PLANE
                tgt = nxt if d == 0 else prv
                rdma = pltpu.make_async_remote_copy(
                    src_ref=out_ref.at[chunk_rows(d, c_send), :],
                    dst_ref=comm3.at[d, j],
                    send_sem=s3s.at[d, j],
                    recv_sem=r3s.at[d, j],
                    device_id=(tgt,),
                    device_id_type=pl.DeviceIdType.MESH,
                )
                rdma.start()
                rdmas.append(rdma)
            for d, rdma in enumerate(rdmas):
                rdma.wait()
                c_recv = (q - j) % PLANE if d == 0 else (q + j) % PLANE
                out_ref[chunk_rows(d, c_recv), :] = comm3[d, j]

    return pl.pallas_call(
        body,
        out_shape=jax.ShapeDtypeStruct((rows, cols), x.dtype),
        in_specs=[pl.BlockSpec(memory_space=pltpu.VMEM)],
        out_specs=pl.BlockSpec(memory_space=pltpu.VMEM),
        scratch_shapes=[
            pltpu.VMEM((2, PLANE - 1, ch, cols), x.dtype),
            pltpu.VMEM((2, 2 * (NZ - 1), sub, cols), x.dtype),
            pltpu.VMEM((2, PLANE - 1, ch, cols), x.dtype),
            pltpu.SemaphoreType.DMA((2, PLANE - 1)),
            pltpu.SemaphoreType.DMA((2, PLANE - 1)),
            pltpu.SemaphoreType.DMA((2, 2 * (NZ - 1))),
            pltpu.SemaphoreType.DMA((2, 2 * (NZ - 1))),
            pltpu.SemaphoreType.DMA((2, PLANE - 1)),
            pltpu.SemaphoreType.DMA((2, PLANE - 1)),
        ],
        compiler_params=pltpu.CompilerParams(collective_id=collective_id),
    )(x)


def kernel(x, Wq, Wk, Wv, Wo, t_emb, W_mod, W_ff1, W_ff2):
    B, S, D = x.shape
    eps = 1e-5
    Dh = 128
    H = Wq.shape[1] // Dh

    mod = t_emb @ W_mod
    sa, sha, ga, sm, shm, gm = jnp.split(mod, 6, axis=-1)

    def ln(h):
        m = h.mean(axis=-1, keepdims=True)
        v = h.var(axis=-1, keepdims=True)
        return (h - m) * lax.rsqrt(v + eps)

    x0 = x
    xm = ln(x0) * (1.0 + sa[:, None, :]) + sha[:, None, :]

    Q = (xm @ Wq).reshape(B, S, H, Dh)
    K = (xm @ Wk).reshape(B, S, H, Dh)
    V = (xm @ Wv).reshape(B, S, H, Dh)
    scores = jnp.einsum("bihd,bjhd->bhij", Q, K) * 0.08838834764831843
    p = jax.nn.softmax(scores, axis=-1)
    o = jnp.einsum("bhij,bjhd->bihd", p, V).reshape(B, S, H * Dh)
    attn_partial = o @ Wo

    attn = _hier_allreduce(
        attn_partial.reshape(B * S, D), collective_id=0
    ).reshape(B, S, D)

    x1 = x0 + ga[:, None, :] * attn
    xm2 = ln(x1) * (1.0 + sm[:, None, :]) + shm[:, None, :]
    h = xm2 @ W_ff1
    h = h * jax.nn.sigmoid(h)
    ffn_partial = h @ W_ff2

    ffn = _hier_allreduce(
        ffn_partial.reshape(B * S, D), collective_id=1
    ).reshape(B, S, D)

    return x1 + gm[:, None, :] * ffn
